# Initial kernel scaffold; baseline (speedup 1.0000x reference)
import jax
import jax.numpy as jnp
from jax.experimental import pallas as pl


def kernel(x, edge_index, ppmi_edge_index, ppmi_edge_attr, W1, as1, ad1, b1, W2, as2, ad2, b2, prelu_l, Wp1, asp1, adp1, We1, ae1, bp1, Wp2, asp2, adp2, We2, ae2, bp2, prelu_g, Wa, ba, Wc, bc):
    raise NotImplementedError("write your pallas kernel here")


# speedup vs baseline: 9.8722x; 9.8722x over previous
import functools

import jax
import jax.numpy as jnp
from jax import lax
from jax.experimental import pallas as pl
from jax.experimental.pallas import tpu as pltpu
from jax.experimental.pallas import tpu_sc as plsc

N = 10000
E = 320000
D = 128
H0 = 128
H1 = 64
OUT = 10

NC = 2
NS = 16
NW = NC * NS
EPT = E // NW
K = 80
NCHUNK = EPT // K
NP = 10240
RPT = NP // NS
ZR = 80

BN = 1000


def _make_sc_agg(H, use_bias):
  HA = H + 16
  mesh = plsc.VectorSubcoreMesh(
      core_axis_name="c", subcore_axis_name="s", num_cores=NC, num_subcores=NS)

  def body(*refs):
    if use_bias:
      (h_hbm, src_hbm, dst_hbm, bias_hbm, s_hbm, d_hbm, g_hbm, z_hbm,
       acc_out,
       s_v, d_v, g_v, src_v, dst_v, bias_v, rows_v, sem,
       acc_sh) = refs
    else:
      (h_hbm, src_hbm, dst_hbm, s_hbm, d_hbm, g_hbm, z_hbm,
       acc_out,
       s_v, d_v, g_v, src_v, dst_v, rows_v, sem,
       acc_sh) = refs
      bias_hbm = bias_v = None

    cid = lax.axis_index("c")
    tid = lax.axis_index("s")
    wid = cid * NS + tid

    pltpu.sync_copy(s_hbm, s_v)
    pltpu.sync_copy(d_hbm, d_v)
    pltpu.sync_copy(g_hbm, g_v)
    pltpu.sync_copy(z_hbm, rows_v)
    for i in range(RPT // ZR):
      pltpu.sync_copy(rows_v, acc_sh.at[pl.ds(tid * RPT + i * ZR, ZR), :])
    plsc.subcore_barrier()

    gvec = g_v[...]
    ebase = wid * EPT

    def chunk(cix, carry):
      base = ebase + cix * K
      pltpu.sync_copy(src_hbm.at[pl.ds(base, K)], src_v)
      pltpu.sync_copy(dst_hbm.at[pl.ds(base, K)], dst_v)
      if use_bias:
        pltpu.sync_copy(bias_hbm.at[pl.ds(base, K)], bias_v)
      pltpu.sync_copy(h_hbm.at[src_v], rows_v)
      rid = jnp.arange(16, dtype=jnp.int32)
      for gi in range(K // 16):
        sl = pl.ds(gi * 16, 16)
        a = plsc.load_gather(s_v, [src_v[sl]]) + plsc.load_gather(d_v, [dst_v[sl]])
        if use_bias:
          a = a + bias_v[sl]
        a = jnp.where(a >= 0, a, 0.2 * a) - gvec
        ex16 = jnp.exp(a)
        rows16 = rid + (16 * gi)
        for c in range(HA):
          colv = jnp.full((16,), c, jnp.int32)
          v = plsc.load_gather(rows_v, [rows16, colv])
          plsc.store_scatter(rows_v, [rows16, colv], v * ex16)
      pltpu.sync_copy(rows_v, acc_sh.at[dst_v], add=True)
      return carry

    lax.fori_loop(0, NCHUNK, chunk, 0)
    plsc.subcore_barrier()

    for i in range(RPT // ZR):
      sl = pl.ds(tid * RPT + i * ZR, ZR)
      pltpu.sync_copy(acc_sh.at[sl, :], rows_v)
      pltpu.sync_copy(rows_v, acc_out.at[cid, sl, :])

  scratch = [
      pltpu.VMEM((N,), jnp.float32),
      pltpu.VMEM((N,), jnp.float32),
      pltpu.VMEM((16,), jnp.float32),
      pltpu.VMEM((K,), jnp.int32),
      pltpu.VMEM((K,), jnp.int32),
  ]
  if use_bias:
    scratch.append(pltpu.VMEM((K,), jnp.float32))
  scratch += [
      pltpu.VMEM((K, HA), jnp.float32),
      pltpu.SemaphoreType.DMA,
      pltpu.VMEM_SHARED((NP, HA), jnp.float32),
  ]

  return pl.kernel(
      body,
      out_type=jax.ShapeDtypeStruct((NC, NP, HA), jnp.float32),
      mesh=mesh,
      scratch_types=scratch,
      compiler_params=pltpu.CompilerParams(
          needs_layout_passes=False, use_tc_tiling_on_sc=False),
  )


_sc_agg = {(H, ub): _make_sc_agg(H, ub)
           for H in (H0, H1) for ub in (False, True)}


def _pro_body(H, x_ref, w_ref, asd_ref, ha_ref, sd_ref, m_ref):
  h = jnp.dot(x_ref[...], w_ref[...], preferred_element_type=jnp.float32)
  sd = jnp.dot(h, asd_ref[...], preferred_element_type=jnp.float32)
  ha_ref[...] = jnp.concatenate(
      [h, jnp.ones((h.shape[0], 16), jnp.float32)], axis=1)
  sd_ref[...] = sd
  cur = jnp.max(sd, axis=0)[None, :]

  @pl.when(pl.program_id(0) == 0)
  def _():
    m_ref[...] = cur

  @pl.when(pl.program_id(0) > 0)
  def _():
    m_ref[...] = jnp.maximum(m_ref[...], cur)


def _make_prologue(H):
  HA = H + 16
  return pl.pallas_call(
      functools.partial(_pro_body, H),
      grid=(N // BN,),
      in_specs=[
          pl.BlockSpec((BN, D), lambda i: (i, 0)),
          pl.BlockSpec((D, H), lambda i: (0, 0)),
          pl.BlockSpec((H, 2), lambda i: (0, 0)),
      ],
      out_specs=[
          pl.BlockSpec((BN, HA), lambda i: (i, 0)),
          pl.BlockSpec((BN, 2), lambda i: (i, 0)),
          pl.BlockSpec((1, 2), lambda i: (0, 0)),
      ],
      out_shape=[
          jax.ShapeDtypeStruct((N, HA), jnp.float32),
          jax.ShapeDtypeStruct((N, 2), jnp.float32),
          jax.ShapeDtypeStruct((1, 2), jnp.float32),
      ],
  )


def _fin_body(H, a0_ref, a1_ref, ha_ref, sd_ref, gbl_ref, b_ref, p_ref, f_ref):
  sd = sd_ref[...]
  gbl = gbl_ref[...]
  g = gbl[0, 0]
  bl = gbl[0, 1]
  al = sd[:, :1] + sd[:, 1:2] + bl
  al = jnp.where(al >= 0, al, 0.2 * al) - g
  exl = jnp.exp(al)
  acc = a0_ref[...] + a1_ref[...]
  h = ha_ref[...][:, :H]
  num = acc[:, :H] + exl * h
  den = acc[:, H:H + 1] + exl + 1e-16
  o = num / den + b_ref[...]
  p = p_ref[0, 0]
  f_ref[...] = jnp.where(o >= 0, o, p * o)


def _make_finish(H):
  HA = H + 16
  return pl.pallas_call(
      functools.partial(_fin_body, H),
      grid=(N // BN,),
      in_specs=[
          pl.BlockSpec((BN, HA), lambda i: (i, 0)),
          pl.BlockSpec((BN, HA), lambda i: (i, 0)),
          pl.BlockSpec((BN, HA), lambda i: (i, 0)),
          pl.BlockSpec((BN, 2), lambda i: (i, 0)),
          pl.BlockSpec((1, 2), lambda i: (0, 0)),
          pl.BlockSpec((1, H), lambda i: (0, 0)),
          pl.BlockSpec((1, 1), lambda i: (0, 0)),
      ],
      out_specs=pl.BlockSpec((BN, H), lambda i: (i, 0)),
      out_shape=jax.ShapeDtypeStruct((N, H), jnp.float32),
  )


def _attr_body(ea_ref, we1_ref, ae1_ref, we2_ref, ae2_ref,
               b1_ref, b2_ref, c_ref, mea_ref, bmax_ref):
  c1 = jnp.sum(we1_ref[...] * ae1_ref[...])
  c2 = jnp.sum(we2_ref[...] * ae2_ref[...])
  ea = ea_ref[...]
  bb1 = ea * c1
  bb2 = ea * c2
  b1_ref[...] = bb1
  b2_ref[...] = bb2
  c_ref[...] = jnp.stack([c1, c2])[None, :]
  mea_ref[...] = jnp.sum(ea).reshape(1, 1)
  bmax_ref[...] = jnp.stack([jnp.max(bb1), jnp.max(bb2)])[None, :]


_EROWS = E // 512
_attr_prep = pl.pallas_call(
    _attr_body,
    out_shape=[
        jax.ShapeDtypeStruct((_EROWS, 512), jnp.float32),
        jax.ShapeDtypeStruct((_EROWS, 512), jnp.float32),
        jax.ShapeDtypeStruct((1, 2), jnp.float32),
        jax.ShapeDtypeStruct((1, 1), jnp.float32),
        jax.ShapeDtypeStruct((1, 2), jnp.float32),
    ],
)


def _fuse_body(l_ref, g_ref, wa_ref, ba_ref, wc_ref, bc_ref, emb_ref, pred_ref):
  l = l_ref[...]
  gg = g_ref[...]
  wa = wa_ref[...]
  ba = ba_ref[0, 0]
  sl = jnp.sum(l * wa, axis=1, keepdims=True) + ba
  sg = jnp.sum(gg * wa, axis=1, keepdims=True) + ba
  m = jnp.maximum(sl, sg)
  el = jnp.exp(sl - m)
  eg = jnp.exp(sg - m)
  emb = (el * l + eg * gg) / (el + eg)
  emb_ref[...] = emb
  pred_ref[...] = jnp.dot(emb, wc_ref[...],
                          preferred_element_type=jnp.float32) + bc_ref[...]


_fuse = pl.pallas_call(
    _fuse_body,
    grid=(N // BN,),
    in_specs=[
        pl.BlockSpec((BN, H1), lambda i: (i, 0)),
        pl.BlockSpec((BN, H1), lambda i: (i, 0)),
        pl.BlockSpec((1, H1), lambda i: (0, 0)),
        pl.BlockSpec((1, 1), lambda i: (0, 0)),
        pl.BlockSpec((H1, OUT), lambda i: (0, 0)),
        pl.BlockSpec((1, OUT), lambda i: (0, 0)),
    ],
    out_specs=[
        pl.BlockSpec((BN, H1), lambda i: (i, 0)),
        pl.BlockSpec((BN, OUT), lambda i: (i, 0)),
    ],
    out_shape=[
        jax.ShapeDtypeStruct((N, H1), jnp.float32),
        jax.ShapeDtypeStruct((N, OUT), jnp.float32),
    ],
)


def _gat_layer(x, src, dst, W, a_s, a_d, b, p, H, bias_e=None, bl=None,
               bmax=None):
  pro = _make_prologue_cache[H]
  asd = jnp.stack([a_s, a_d], axis=1)
  ha, sd, m = pro(x, W, asd)
  if bias_e is None:
    bl = jnp.float32(0.0)
    bmax = jnp.float32(0.0)
  gb = m[0, 0] + m[0, 1] + bmax
  gmax = jnp.where(gb >= 0, gb, 0.2 * gb)
  g16 = jnp.broadcast_to(gmax.reshape(1), (16,))
  s = sd[:, 0]
  d = sd[:, 1]
  HA = H + 16
  z = jnp.zeros((ZR, HA), jnp.float32)
  agg = _sc_agg[(H, bias_e is not None)]
  if bias_e is None:
    acc = agg(ha, src, dst, s, d, g16, z)
  else:
    acc = agg(ha, src, dst, bias_e, s, d, g16, z)
  acc = acc[:, :N]
  gbl = jnp.stack([gmax, bl]).reshape(1, 2)
  fin = _make_finish_cache[H]
  return fin(acc[0], acc[1], ha, sd, gbl, b.reshape(1, H), p.reshape(1, 1))


_make_prologue_cache = {H: _make_prologue(H) for H in (H0, H1)}
_make_finish_cache = {H: _make_finish(H) for H in (H0, H1)}


def kernel(x, edge_index, ppmi_edge_index, ppmi_edge_attr,
           W1, as1, ad1, b1, W2, as2, ad2, b2, prelu_l,
           Wp1, asp1, adp1, We1, ae1, bp1,
           Wp2, asp2, adp2, We2, ae2, bp2, prelu_g,
           Wa, ba, Wc, bc):
  src, dst = edge_index[0], edge_index[1]
  ps, pd = ppmi_edge_index[0], ppmi_edge_index[1]

  ea2 = ppmi_edge_attr.reshape(_EROWS, 512)
  bias1m, bias2m, c, meas, bmaxo = _attr_prep(
      ea2, We1, ae1.reshape(1, H0), We2.reshape(1, H1), ae2.reshape(1, H1))
  mea = meas[0, 0] / E
  bl1 = mea * c[0, 0]
  bl2 = mea * c[0, 1]
  bmax1 = jnp.maximum(bmaxo[0, 0], bl1)
  bmax2 = jnp.maximum(bmaxo[0, 1], bl2)
  bias1 = bias1m.reshape(E)
  bias2 = bias2m.reshape(E)

  f1 = _gat_layer(x, src, dst, W1, as1, ad1, b1, prelu_l, H0)
  l_out = _gat_layer(f1, src, dst, W2, as2, ad2, b2, prelu_l, H1)
  g1 = _gat_layer(x, ps, pd, Wp1, asp1, adp1, bp1, prelu_g, H0,
                  bias_e=bias1, bl=bl1, bmax=bmax1)
  g_out = _gat_layer(g1, ps, pd, Wp2, asp2, adp2, bp2, prelu_g, H1,
                     bias_e=bias2, bl=bl2, bmax=bmax2)

  emb, pred = _fuse(l_out, g_out, Wa.reshape(1, H1), ba.reshape(1, 1),
                    Wc, bc.reshape(1, OUT))
  return (emb, pred)


# speedup vs baseline: 10.4304x; 1.0565x over previous
import functools

import jax
import jax.numpy as jnp
from jax import lax
from jax.experimental import pallas as pl
from jax.experimental.pallas import tpu as pltpu
from jax.experimental.pallas import tpu_sc as plsc

N = 10000
E = 320000
D = 128
H0 = 128
H1 = 64
OUT = 10

NC = 2
NS = 16
NW = NC * NS
K = 128
NCHUNK = 79
EPT = NCHUNK * K
EPAD = NW * EPT
NP = 10112
RPT = NP // NS
ZCH = ((0, 120), (120, 120), (240, 120), (360, 120), (480, 120), (600, 32))

BN = 1000


def _make_sc_agg(H):
  HA = H + 16
  NB = 1 if H == H0 else 2
  mesh = plsc.VectorSubcoreMesh(
      core_axis_name="c", subcore_axis_name="s", num_cores=NC, num_subcores=NS)

  def body(h_hbm, src_hbm, dst_hbm, bias_hbm, s_hbm, d_hbm, g_hbm, z_hbm,
           acc_out, s_v, d_v, g_v, *rest):
    bufs = rest[:4 * NB]
    sems = rest[4 * NB:4 * NB + NB]
    acc_sh = rest[-1]

    cid = lax.axis_index("c")
    tid = lax.axis_index("s")
    wid = cid * NS + tid

    pltpu.sync_copy(s_hbm, s_v)
    pltpu.sync_copy(d_hbm, d_v)
    pltpu.sync_copy(g_hbm, g_v)
    rows0 = bufs[3]
    pltpu.sync_copy(z_hbm, rows0.at[pl.ds(0, 120)])
    for off, cn in ZCH:
      pltpu.sync_copy(rows0.at[pl.ds(0, cn)],
                      acc_sh.at[pl.ds(tid * RPT + off, cn), :])
    plsc.subcore_barrier()

    gvec = g_v[...]
    ebase = wid * EPT
    rid = jnp.arange(16, dtype=jnp.int32)

    def load_idx(base, b):
      src_v, dst_v, bias_v, _ = bufs[4 * b:4 * b + 4]
      pltpu.sync_copy(src_hbm.at[pl.ds(base, K)], src_v)
      pltpu.sync_copy(dst_hbm.at[pl.ds(base, K)], dst_v)
      pltpu.sync_copy(bias_hbm.at[pl.ds(base, K)], bias_v)

    def scale_and_scatter(b):
      src_v, dst_v, bias_v, rows_v = bufs[4 * b:4 * b + 4]
      for gi in range(K // 16):
        sl = pl.ds(gi * 16, 16)
        a = plsc.load_gather(s_v, [src_v[sl]]) + plsc.load_gather(d_v, [dst_v[sl]])
        a = a + bias_v[sl]
        a = jnp.where(a >= 0, a, 0.2 * a) - gvec
        ex16 = jnp.exp(a)
        rows16 = rid + (16 * gi)
        for c in range(H + 1):
          colv = jnp.full((16,), c, jnp.int32)
          v = plsc.load_gather(rows_v, [rows16, colv])
          plsc.store_scatter(rows_v, [rows16, colv], v * ex16)
      pltpu.sync_copy(rows_v, acc_sh.at[dst_v], add=True)

    if NB == 1:
      def chunk(cix, carry):
        base = ebase + cix * K
        load_idx(base, 0)
        pltpu.sync_copy(h_hbm.at[bufs[0]], bufs[3])
        scale_and_scatter(0)
        return carry
      lax.fori_loop(0, NCHUNK, chunk, 0)
    else:
      def pair(pi, carry):
        base = ebase + pi * (2 * K)
        load_idx(base, 0)
        cp0 = pltpu.async_copy(h_hbm.at[bufs[0]], bufs[3], sems[0])
        load_idx(base + K, 1)
        cp1 = pltpu.async_copy(h_hbm.at[bufs[4]], bufs[7], sems[1])
        cp0.wait()
        scale_and_scatter(0)
        cp1.wait()
        scale_and_scatter(1)
        return carry
      lax.fori_loop(0, NCHUNK // 2, pair, 0)
      load_idx(ebase + (NCHUNK - 1) * K, 0)
      pltpu.sync_copy(h_hbm.at[bufs[0]], bufs[3])
      scale_and_scatter(0)

    plsc.subcore_barrier()

    for off, cn in ZCH:
      sl = pl.ds(tid * RPT + off, cn)
      pltpu.sync_copy(acc_sh.at[sl, :], rows0.at[pl.ds(0, cn)])
      pltpu.sync_copy(rows0.at[pl.ds(0, cn)], acc_out.at[cid, sl, :])

  scratch = [
      pltpu.VMEM((N,), jnp.float32),
      pltpu.VMEM((N,), jnp.float32),
      pltpu.VMEM((16,), jnp.float32),
  ]
  for _ in range(NB):
    scratch += [
        pltpu.VMEM((K,), jnp.int32),
        pltpu.VMEM((K,), jnp.int32),
        pltpu.VMEM((K,), jnp.float32),
        pltpu.VMEM((K, HA), jnp.float32),
    ]
  scratch += [pltpu.SemaphoreType.DMA] * NB
  scratch += [pltpu.VMEM_SHARED((NP, HA), jnp.float32)]

  return pl.kernel(
      body,
      out_type=jax.ShapeDtypeStruct((NC, NP, HA), jnp.float32),
      mesh=mesh,
      scratch_types=scratch,
      compiler_params=pltpu.CompilerParams(
          needs_layout_passes=False, use_tc_tiling_on_sc=False),
  )


_sc_agg = {H: _make_sc_agg(H) for H in (H0, H1)}


def _pro_body(H, x_ref, w_ref, asd_ref, ha_ref, sd_ref, m_ref):
  h = jnp.dot(x_ref[...], w_ref[...], preferred_element_type=jnp.float32)
  sd = jnp.dot(h, asd_ref[...], preferred_element_type=jnp.float32)
  ha_ref[...] = jnp.concatenate(
      [h, jnp.ones((h.shape[0], 16), jnp.float32)], axis=1)
  sd_ref[...] = sd
  cur = jnp.max(sd, axis=0)[None, :]

  @pl.when(pl.program_id(0) == 0)
  def _():
    m_ref[...] = cur

  @pl.when(pl.program_id(0) > 0)
  def _():
    m_ref[...] = jnp.maximum(m_ref[...], cur)


def _make_prologue(H):
  HA = H + 16
  return pl.pallas_call(
      functools.partial(_pro_body, H),
      grid=(N // BN,),
      in_specs=[
          pl.BlockSpec((BN, D), lambda i: (i, 0)),
          pl.BlockSpec((D, H), lambda i: (0, 0)),
          pl.BlockSpec((H, 2), lambda i: (0, 0)),
      ],
      out_specs=[
          pl.BlockSpec((BN, HA), lambda i: (i, 0)),
          pl.BlockSpec((BN, 2), lambda i: (i, 0)),
          pl.BlockSpec((1, 2), lambda i: (0, 0)),
      ],
      out_shape=[
          jax.ShapeDtypeStruct((N, HA), jnp.float32),
          jax.ShapeDtypeStruct((N, 2), jnp.float32),
          jax.ShapeDtypeStruct((1, 2), jnp.float32),
      ],
  )


def _fin_body(H, a0_ref, a1_ref, ha_ref, sd_ref, gbl_ref, b_ref, p_ref, f_ref):
  sd = sd_ref[...]
  gbl = gbl_ref[...]
  g = gbl[0, 0]
  bl = gbl[0, 1]
  al = sd[:, :1] + sd[:, 1:2] + bl
  al = jnp.where(al >= 0, al, 0.2 * al) - g
  exl = jnp.exp(al)
  acc = a0_ref[...] + a1_ref[...]
  h = ha_ref[...][:, :H]
  num = acc[:, :H] + exl * h
  den = acc[:, H:H + 1] + exl + 1e-16
  o = num / den + b_ref[...]
  p = p_ref[0, 0]
  f_ref[...] = jnp.where(o >= 0, o, p * o)


def _make_finish(H):
  HA = H + 16
  return pl.pallas_call(
      functools.partial(_fin_body, H),
      grid=(N // BN,),
      in_specs=[
          pl.BlockSpec((BN, HA), lambda i: (i, 0)),
          pl.BlockSpec((BN, HA), lambda i: (i, 0)),
          pl.BlockSpec((BN, HA), lambda i: (i, 0)),
          pl.BlockSpec((BN, 2), lambda i: (i, 0)),
          pl.BlockSpec((1, 2), lambda i: (0, 0)),
          pl.BlockSpec((1, H), lambda i: (0, 0)),
          pl.BlockSpec((1, 1), lambda i: (0, 0)),
      ],
      out_specs=pl.BlockSpec((BN, H), lambda i: (i, 0)),
      out_shape=jax.ShapeDtypeStruct((N, H), jnp.float32),
  )


def _attr_body(ea_ref, we1_ref, ae1_ref, we2_ref, ae2_ref,
               b1_ref, b2_ref, c_ref, mea_ref, bmax_ref):
  c1 = jnp.sum(we1_ref[...] * ae1_ref[...])
  c2 = jnp.sum(we2_ref[...] * ae2_ref[...])
  ea = ea_ref[...]
  bb1 = ea * c1
  bb2 = ea * c2
  b1_ref[...] = bb1
  b2_ref[...] = bb2
  c_ref[...] = jnp.stack([c1, c2])[None, :]
  mea_ref[...] = jnp.sum(ea).reshape(1, 1)
  bmax_ref[...] = jnp.stack([jnp.max(bb1), jnp.max(bb2)])[None, :]


_EROWS = E // 512
_attr_prep = pl.pallas_call(
    _attr_body,
    out_shape=[
        jax.ShapeDtypeStruct((_EROWS, 512), jnp.float32),
        jax.ShapeDtypeStruct((_EROWS, 512), jnp.float32),
        jax.ShapeDtypeStruct((1, 2), jnp.float32),
        jax.ShapeDtypeStruct((1, 1), jnp.float32),
        jax.ShapeDtypeStruct((1, 2), jnp.float32),
    ],
)


def _fuse_body(l_ref, g_ref, wa_ref, ba_ref, wc_ref, bc_ref, emb_ref, pred_ref):
  l = l_ref[...]
  gg = g_ref[...]
  wa = wa_ref[...]
  ba = ba_ref[0, 0]
  sl = jnp.sum(l * wa, axis=1, keepdims=True) + ba
  sg = jnp.sum(gg * wa, axis=1, keepdims=True) + ba
  m = jnp.maximum(sl, sg)
  el = jnp.exp(sl - m)
  eg = jnp.exp(sg - m)
  emb = (el * l + eg * gg) / (el + eg)
  emb_ref[...] = emb
  pred_ref[...] = jnp.dot(emb, wc_ref[...],
                          preferred_element_type=jnp.float32) + bc_ref[...]


_fuse = pl.pallas_call(
    _fuse_body,
    grid=(N // BN,),
    in_specs=[
        pl.BlockSpec((BN, H1), lambda i: (i, 0)),
        pl.BlockSpec((BN, H1), lambda i: (i, 0)),
        pl.BlockSpec((1, H1), lambda i: (0, 0)),
        pl.BlockSpec((1, 1), lambda i: (0, 0)),
        pl.BlockSpec((H1, OUT), lambda i: (0, 0)),
        pl.BlockSpec((1, OUT), lambda i: (0, 0)),
    ],
    out_specs=[
        pl.BlockSpec((BN, H1), lambda i: (i, 0)),
        pl.BlockSpec((BN, OUT), lambda i: (i, 0)),
    ],
    out_shape=[
        jax.ShapeDtypeStruct((N, H1), jnp.float32),
        jax.ShapeDtypeStruct((N, OUT), jnp.float32),
    ],
)


def _gat_layer(x, srcp, dstp, biasp, W, a_s, a_d, b, p, H, bl, bmax):
  pro = _make_prologue_cache[H]
  asd = jnp.stack([a_s, a_d], axis=1)
  ha, sd, m = pro(x, W, asd)
  gb = m[0, 0] + m[0, 1] + bmax
  gmax = jnp.where(gb >= 0, gb, 0.2 * gb)
  g16 = jnp.broadcast_to(gmax.reshape(1), (16,))
  s = sd[:, 0]
  d = sd[:, 1]
  HA = H + 16
  z = jnp.zeros((120, HA), jnp.float32)
  acc = _sc_agg[H](ha, srcp, dstp, biasp, s, d, g16, z)
  acc = acc[:, :N]
  gbl = jnp.stack([gmax, bl]).reshape(1, 2)
  fin = _make_finish_cache[H]
  return fin(acc[0], acc[1], ha, sd, gbl, b.reshape(1, H), p.reshape(1, 1))


_make_prologue_cache = {H: _make_prologue(H) for H in (H0, H1)}
_make_finish_cache = {H: _make_finish(H) for H in (H0, H1)}


def kernel(x, edge_index, ppmi_edge_index, ppmi_edge_attr,
           W1, as1, ad1, b1, W2, as2, ad2, b2, prelu_l,
           Wp1, asp1, adp1, We1, ae1, bp1,
           Wp2, asp2, adp2, We2, ae2, bp2, prelu_g,
           Wa, ba, Wc, bc):
  src, dst = edge_index[0], edge_index[1]
  ps, pd = ppmi_edge_index[0], ppmi_edge_index[1]

  ea2 = ppmi_edge_attr.reshape(_EROWS, 512)
  bias1m, bias2m, c, meas, bmaxo = _attr_prep(
      ea2, We1, ae1.reshape(1, H0), We2.reshape(1, H1), ae2.reshape(1, H1))
  mea = meas[0, 0] / E
  bl1 = mea * c[0, 0]
  bl2 = mea * c[0, 1]
  bmax1 = jnp.maximum(bmaxo[0, 0], bl1)
  bmax2 = jnp.maximum(bmaxo[0, 1], bl2)

  padi = jnp.zeros((EPAD - E,), jnp.int32)
  padb = jnp.full((EPAD - E,), -1e30, jnp.float32)
  srcp = jnp.concatenate([src, padi])
  dstp = jnp.concatenate([dst, padi])
  psp = jnp.concatenate([ps, padi])
  pdp = jnp.concatenate([pd, padi])
  bias0 = jnp.concatenate([jnp.zeros((E,), jnp.float32), padb])
  bias1 = jnp.concatenate([bias1m.reshape(E), padb])
  bias2 = jnp.concatenate([bias2m.reshape(E), padb])
  zf = jnp.float32(0.0)

  f1 = _gat_layer(x, srcp, dstp, bias0, W1, as1, ad1, b1, prelu_l, H0, zf, zf)
  l_out = _gat_layer(f1, srcp, dstp, bias0, W2, as2, ad2, b2, prelu_l, H1,
                     zf, zf)
  g1 = _gat_layer(x, psp, pdp, bias1, Wp1, asp1, adp1, bp1, prelu_g, H0,
                  bl1, bmax1)
  g_out = _gat_layer(g1, psp, pdp, bias2, Wp2, asp2, adp2, bp2, prelu_g, H1,
                     bl2, bmax2)

  emb, pred = _fuse(l_out, g_out, Wa.reshape(1, H1), ba.reshape(1, 1),
                    Wc, bc.reshape(1, OUT))
  return (emb, pred)
